# baseline (speedup 1.0000x reference)
import functools

import jax
import jax.numpy as jnp
from jax import lax
from jax.experimental import pallas as pl
from jax.experimental.pallas import tpu as pltpu
from jax.experimental.pallas import tpu_sc as plsc


def _sc_gather_users(user_id, user_table):
    B = user_id.shape[0]
    E = user_table.shape[1]
    info = plsc.get_sparse_core_info()
    NW = info.num_cores * info.num_subcores
    bpw = B // NW
    mesh = plsc.VectorSubcoreMesh(core_axis_name="c", subcore_axis_name="s")

    @functools.partial(
        pl.kernel,
        mesh=mesh,
        compiler_params=pltpu.CompilerParams(use_tc_tiling_on_sc=False),
        out_type=jax.ShapeDtypeStruct((B, E), jnp.float32),
        scratch_types=[
            pltpu.VMEM((bpw,), jnp.int32),
            pltpu.VMEM((bpw, E), jnp.float32),
            pltpu.SemaphoreType.DMA,
            pltpu.SemaphoreType.DMA,
        ],
    )
    def gather_kernel(uid_hbm, table_hbm, out_hbm, idx_v, rows_v,
                      sem_i, sem_g):
        wid = lax.axis_index("s") * info.num_cores + lax.axis_index("c")
        base = wid * bpw
        pltpu.async_copy(uid_hbm.at[pl.ds(base, bpw)], idx_v, sem_i).wait()
        copies = []
        for c in range(bpw // 16):
            vec = idx_v[pl.ds(c * 16, 16)]
            for j in range(16):
                i = c * 16 + j
                copies.append(pltpu.async_copy(
                    table_hbm.at[pl.ds(vec[j], 1)],
                    rows_v.at[pl.ds(i, 1)], sem_g))
        for c in copies:
            c.wait()
        pltpu.sync_copy(rows_v, out_hbm.at[pl.ds(base, bpw)])

    return gather_kernel(user_id, user_table)


def _tc_combine_t(u, time_r, dow_r, tt_t, dt_t, w_t, b_c):
    B, EU = u.shape
    ET, TV = tt_t.shape
    DV = dt_t.shape[1]
    N = w_t.shape[0]
    BN = 512

    def body(u_ref, t_ref, d_ref, tt_ref, dt_ref, w_ref, b_ref, o_ref):
        t_oh = (lax.broadcasted_iota(jnp.int32, (TV, BN), 0)
                == t_ref[...]).astype(jnp.float32)
        d_oh = (lax.broadcasted_iota(jnp.int32, (DV, BN), 0)
                == d_ref[...]).astype(jnp.float32)
        proj_t = jnp.dot(w_ref[:, EU:EU + ET], tt_ref[...],
                         preferred_element_type=jnp.float32)
        proj_d = jnp.dot(w_ref[:, EU + ET:EU + 2 * ET], dt_ref[...],
                         preferred_element_type=jnp.float32)
        acc = lax.dot_general(w_ref[:, 0:EU], u_ref[...],
                              (((1,), (1,)), ((), ())),
                              preferred_element_type=jnp.float32)
        acc += jnp.dot(proj_t, t_oh, preferred_element_type=jnp.float32)
        acc += jnp.dot(proj_d, d_oh, preferred_element_type=jnp.float32)
        o_ref[...] = acc + b_ref[...]

    return pl.pallas_call(
        body,
        grid=(B // BN,),
        in_specs=[
            pl.BlockSpec((BN, EU), lambda i: (i, 0)),
            pl.BlockSpec((1, BN), lambda i: (0, i)),
            pl.BlockSpec((1, BN), lambda i: (0, i)),
            pl.BlockSpec((ET, TV), lambda i: (0, 0)),
            pl.BlockSpec((ET, DV), lambda i: (0, 0)),
            pl.BlockSpec(w_t.shape, lambda i: (0, 0)),
            pl.BlockSpec((N, 1), lambda i: (0, 0)),
        ],
        out_specs=pl.BlockSpec((N, BN), lambda i: (0, i)),
        out_shape=jax.ShapeDtypeStruct((N, B), jnp.float32),
    )(u, time_r, dow_r, tt_t, dt_t, w_t, b_c)


def kernel(user_id, time, day_of_week, user_table, time_table, dow_table, W, b):
    u = _sc_gather_users(user_id, user_table)
    out_t = _tc_combine_t(u, time.reshape(1, -1), day_of_week.reshape(1, -1),
                          time_table.T, dow_table.T, W.T, b.reshape(-1, 1))
    return out_t.T


# speedup vs baseline: 1.4679x; 1.4679x over previous
import functools

import jax
import jax.numpy as jnp
from jax import lax
from jax.experimental import pallas as pl
from jax.experimental.pallas import tpu as pltpu
from jax.experimental.pallas import tpu_sc as plsc


def _sc_gather_users(user_id, user_table):
    B = user_id.shape[0]
    E = user_table.shape[1]
    info = plsc.get_sparse_core_info()
    NW = info.num_cores * info.num_subcores
    bpw = B // NW
    mesh = plsc.VectorSubcoreMesh(core_axis_name="c", subcore_axis_name="s")

    @functools.partial(
        pl.kernel,
        mesh=mesh,
        compiler_params=pltpu.CompilerParams(use_tc_tiling_on_sc=True),
        out_type=jax.ShapeDtypeStruct((B, E), jnp.float32),
        scratch_types=[
            pltpu.VMEM((bpw,), jnp.int32),
            pltpu.VMEM((bpw, E), jnp.float32),
            pltpu.SemaphoreType.DMA,
            pltpu.SemaphoreType.DMA,
        ],
    )
    def gather_kernel(uid_hbm, table_hbm, out_hbm, idx_v, rows_v,
                      sem_i, sem_g):
        wid = lax.axis_index("s") * info.num_cores + lax.axis_index("c")
        base = wid * bpw
        pltpu.async_copy(uid_hbm.at[pl.ds(base, bpw)], idx_v, sem_i).wait()
        copies = []
        for c in range(bpw // 16):
            vec = idx_v[pl.ds(c * 16, 16)]
            for j in range(16):
                i = c * 16 + j
                copies.append(pltpu.async_copy(
                    table_hbm.at[pl.ds(vec[j], 1)],
                    rows_v.at[pl.ds(i, 1)], sem_g))
        for c in copies:
            c.wait()
        pltpu.sync_copy(rows_v, out_hbm.at[pl.ds(base, bpw)])

    return gather_kernel(user_id, user_table)


def _tc_combine_t(u, time_r, dow_r, tt_t, dt_t, w_t, b_c):
    B, EU = u.shape
    ET, TV = tt_t.shape
    DV = dt_t.shape[1]
    N = w_t.shape[0]
    BN = 512

    def body(u_ref, t_ref, d_ref, tt_ref, dt_ref, w_ref, b_ref, o_ref):
        t_oh = (lax.broadcasted_iota(jnp.int32, (TV, BN), 0)
                == t_ref[...]).astype(jnp.float32)
        d_oh = (lax.broadcasted_iota(jnp.int32, (DV, BN), 0)
                == d_ref[...]).astype(jnp.float32)
        proj_t = jnp.dot(w_ref[:, EU:EU + ET], tt_ref[...],
                         preferred_element_type=jnp.float32)
        proj_d = jnp.dot(w_ref[:, EU + ET:EU + 2 * ET], dt_ref[...],
                         preferred_element_type=jnp.float32)
        acc = lax.dot_general(w_ref[:, 0:EU], u_ref[...],
                              (((1,), (1,)), ((), ())),
                              preferred_element_type=jnp.float32)
        acc += jnp.dot(proj_t, t_oh, preferred_element_type=jnp.float32)
        acc += jnp.dot(proj_d, d_oh, preferred_element_type=jnp.float32)
        o_ref[...] = acc + b_ref[...]

    return pl.pallas_call(
        body,
        grid=(B // BN,),
        in_specs=[
            pl.BlockSpec((BN, EU), lambda i: (i, 0)),
            pl.BlockSpec((1, BN), lambda i: (0, i)),
            pl.BlockSpec((1, BN), lambda i: (0, i)),
            pl.BlockSpec((ET, TV), lambda i: (0, 0)),
            pl.BlockSpec((ET, DV), lambda i: (0, 0)),
            pl.BlockSpec(w_t.shape, lambda i: (0, 0)),
            pl.BlockSpec((N, 1), lambda i: (0, 0)),
        ],
        out_specs=pl.BlockSpec((N, BN), lambda i: (0, i)),
        out_shape=jax.ShapeDtypeStruct((N, B), jnp.float32),
    )(u, time_r, dow_r, tt_t, dt_t, w_t, b_c)


def kernel(user_id, time, day_of_week, user_table, time_table, dow_table, W, b):
    u = _sc_gather_users(user_id, user_table)
    out_t = _tc_combine_t(u, time.reshape(1, -1), day_of_week.reshape(1, -1),
                          time_table.T, dow_table.T, W.T, b.reshape(-1, 1))
    return out_t.T


# speedup vs baseline: 1.5209x; 1.0361x over previous
import functools

import jax
import jax.numpy as jnp
from jax import lax
from jax.experimental import pallas as pl
from jax.experimental.pallas import tpu as pltpu
from jax.experimental.pallas import tpu_sc as plsc


def _sc_gather_users(user_id, user_table):
    B = user_id.shape[0]
    E = user_table.shape[1]
    info = plsc.get_sparse_core_info()
    NW = info.num_cores * info.num_subcores
    bpw = B // NW
    mesh = plsc.VectorSubcoreMesh(core_axis_name="c", subcore_axis_name="s")

    @functools.partial(
        pl.kernel,
        mesh=mesh,
        compiler_params=pltpu.CompilerParams(use_tc_tiling_on_sc=True),
        out_type=jax.ShapeDtypeStruct((B, E), jnp.float32),
        scratch_types=[
            pltpu.VMEM((bpw,), jnp.int32),
            pltpu.VMEM((bpw, E), jnp.float32),
            pltpu.SemaphoreType.DMA,
            pltpu.SemaphoreType.DMA,
        ],
    )
    def gather_kernel(uid_hbm, table_hbm, out_hbm, idx_v, rows_v,
                      sem_i, sem_g):
        wid = lax.axis_index("s") * info.num_cores + lax.axis_index("c")
        base = wid * bpw
        pltpu.async_copy(uid_hbm.at[pl.ds(base, bpw)], idx_v, sem_i).wait()
        copies = []
        for c in range(bpw // 16):
            vec = idx_v[pl.ds(c * 16, 16)]
            for j in range(16):
                i = c * 16 + j
                copies.append(pltpu.async_copy(
                    table_hbm.at[pl.ds(vec[j], 1)],
                    rows_v.at[pl.ds(i, 1)], sem_g))
        for c in copies:
            c.wait()
        pltpu.sync_copy(rows_v, out_hbm.at[pl.ds(base, bpw)])

    return gather_kernel(user_id, user_table)


def _tc_combine_t(u, time_r, dow_r, tt_t, dt_t, w_t, b_c):
    B, EU = u.shape
    ET, TV = tt_t.shape
    DV = dt_t.shape[1]
    N = w_t.shape[0]
    BN = 1024

    def body(u_ref, t_ref, d_ref, tt_ref, dt_ref, w_ref, b_ref, o_ref):
        t_oh = (lax.broadcasted_iota(jnp.int32, (TV, BN), 0)
                == t_ref[...]).astype(jnp.float32)
        d_oh = (lax.broadcasted_iota(jnp.int32, (DV, BN), 0)
                == d_ref[...]).astype(jnp.float32)
        proj_t = jnp.dot(w_ref[:, EU:EU + ET], tt_ref[...],
                         preferred_element_type=jnp.float32)
        proj_d = jnp.dot(w_ref[:, EU + ET:EU + 2 * ET], dt_ref[...],
                         preferred_element_type=jnp.float32)
        acc = lax.dot_general(w_ref[:, 0:EU], u_ref[...],
                              (((1,), (1,)), ((), ())),
                              preferred_element_type=jnp.float32)
        acc += jnp.dot(proj_t, t_oh, preferred_element_type=jnp.float32)
        acc += jnp.dot(proj_d, d_oh, preferred_element_type=jnp.float32)
        o_ref[...] = acc + b_ref[...]

    return pl.pallas_call(
        body,
        grid=(B // BN,),
        in_specs=[
            pl.BlockSpec((BN, EU), lambda i: (i, 0)),
            pl.BlockSpec((1, BN), lambda i: (0, i)),
            pl.BlockSpec((1, BN), lambda i: (0, i)),
            pl.BlockSpec((ET, TV), lambda i: (0, 0)),
            pl.BlockSpec((ET, DV), lambda i: (0, 0)),
            pl.BlockSpec(w_t.shape, lambda i: (0, 0)),
            pl.BlockSpec((N, 1), lambda i: (0, 0)),
        ],
        out_specs=pl.BlockSpec((N, BN), lambda i: (0, i)),
        out_shape=jax.ShapeDtypeStruct((N, B), jnp.float32),
    )(u, time_r, dow_r, tt_t, dt_t, w_t, b_c)


def kernel(user_id, time, day_of_week, user_table, time_table, dow_table, W, b):
    u = _sc_gather_users(user_id, user_table)
    out_t = _tc_combine_t(u, time.reshape(1, -1), day_of_week.reshape(1, -1),
                          time_table.T, dow_table.T, W.T, b.reshape(-1, 1))
    return out_t.T


# speedup vs baseline: 1.5561x; 1.0231x over previous
import functools

import jax
import jax.numpy as jnp
from jax import lax
from jax.experimental import pallas as pl
from jax.experimental.pallas import tpu as pltpu
from jax.experimental.pallas import tpu_sc as plsc


def _sc_gather_users(user_id, user_table):
    B = user_id.shape[0]
    E = user_table.shape[1]
    info = plsc.get_sparse_core_info()
    NW = info.num_cores * info.num_subcores
    L = info.num_lanes
    bpw = B // NW
    mesh = plsc.VectorSubcoreMesh(core_axis_name="c", subcore_axis_name="s")

    @functools.partial(
        pl.kernel,
        mesh=mesh,
        compiler_params=pltpu.CompilerParams(use_tc_tiling_on_sc=True),
        out_type=jax.ShapeDtypeStruct((B, E), jnp.float32),
        scratch_types=[
            pltpu.VMEM((bpw,), jnp.int32),
            pltpu.VMEM((bpw, E), jnp.float32),
            pltpu.SemaphoreType.DMA,
            pltpu.SemaphoreType.DMA,
        ],
    )
    def gather_kernel(uid_hbm, table_hbm, out_hbm, idx_v, rows_v,
                      sem_i, sem_g):
        wid = lax.axis_index("s") * info.num_cores + lax.axis_index("c")
        base = wid * bpw
        pltpu.async_copy(uid_hbm.at[pl.ds(base, bpw)], idx_v, sem_i).wait()

        def issue(c, carry):
            vec = idx_v[pl.ds(c * L, L)]
            for j in range(L):
                pltpu.async_copy(table_hbm.at[pl.ds(vec[j], 1)],
                                 rows_v.at[pl.ds(c * L + j, 1)], sem_g)
            return carry

        def drain(c, carry):
            vec = idx_v[pl.ds(c * L, L)]
            for j in range(L):
                pltpu.make_async_copy(table_hbm.at[pl.ds(vec[j], 1)],
                                      rows_v.at[pl.ds(c * L + j, 1)],
                                      sem_g).wait()
            return carry

        lax.fori_loop(0, bpw // L, issue, 0)
        lax.fori_loop(0, bpw // L, drain, 0)
        pltpu.sync_copy(rows_v, out_hbm.at[pl.ds(base, bpw)])

    return gather_kernel(user_id, user_table)


def _tc_combine_t(u, time_r, dow_r, tt_t, dt_t, w_t, b_c):
    B, EU = u.shape
    ET, TV = tt_t.shape
    DV = dt_t.shape[1]
    N = w_t.shape[0]
    BN = 1024

    def body(u_ref, t_ref, d_ref, tt_ref, dt_ref, w_ref, b_ref, o_ref):
        t_oh = (lax.broadcasted_iota(jnp.int32, (TV, BN), 0)
                == t_ref[...]).astype(jnp.float32)
        d_oh = (lax.broadcasted_iota(jnp.int32, (DV, BN), 0)
                == d_ref[...]).astype(jnp.float32)
        proj_t = jnp.dot(w_ref[:, EU:EU + ET], tt_ref[...],
                         preferred_element_type=jnp.float32)
        proj_d = jnp.dot(w_ref[:, EU + ET:EU + 2 * ET], dt_ref[...],
                         preferred_element_type=jnp.float32)
        acc = lax.dot_general(w_ref[:, 0:EU], u_ref[...],
                              (((1,), (1,)), ((), ())),
                              preferred_element_type=jnp.float32)
        acc += jnp.dot(proj_t, t_oh, preferred_element_type=jnp.float32)
        acc += jnp.dot(proj_d, d_oh, preferred_element_type=jnp.float32)
        o_ref[...] = acc + b_ref[...]

    return pl.pallas_call(
        body,
        grid=(B // BN,),
        in_specs=[
            pl.BlockSpec((BN, EU), lambda i: (i, 0)),
            pl.BlockSpec((1, BN), lambda i: (0, i)),
            pl.BlockSpec((1, BN), lambda i: (0, i)),
            pl.BlockSpec((ET, TV), lambda i: (0, 0)),
            pl.BlockSpec((ET, DV), lambda i: (0, 0)),
            pl.BlockSpec(w_t.shape, lambda i: (0, 0)),
            pl.BlockSpec((N, 1), lambda i: (0, 0)),
        ],
        out_specs=pl.BlockSpec((N, BN), lambda i: (0, i)),
        out_shape=jax.ShapeDtypeStruct((N, B), jnp.float32),
    )(u, time_r, dow_r, tt_t, dt_t, w_t, b_c)


def kernel(user_id, time, day_of_week, user_table, time_table, dow_table, W, b):
    u = _sc_gather_users(user_id, user_table)
    out_t = _tc_combine_t(u, time.reshape(1, -1), day_of_week.reshape(1, -1),
                          time_table.T, dow_table.T, W.T, b.reshape(-1, 1))
    return out_t.T
